# baseline (speedup 1.0000x reference)
import functools

import jax
import jax.numpy as jnp
from jax import lax
from jax.experimental import pallas as pl
from jax.experimental.pallas import tpu as pltpu
from jax.experimental.pallas import tpu_sc as plsc

_NI = 100
_PAIRS = _NI * _NI
_D = 128

_NC = 2
_NS = 16
_NW = _NC * _NS

_CHUNK = 128
_BR = 64
_TB = _BR * 50


_J1B = 10


def _h12_body(g2m_ref, g1t_ref, out_ref):
    lhs = g2m_ref[...]
    w = _D * _J1B
    gc = jnp.repeat(g1t_ref[...], 2, axis=0)
    row_m2 = lax.broadcasted_iota(jnp.int32, (32, 1), 0) % 2
    col_m2 = lax.broadcasted_iota(jnp.int32, (1, w), 1) % 2
    rhs = jnp.where(row_m2 == col_m2, gc, 0.0)
    res = jnp.dot(lhs, rhs, preferred_element_type=jnp.float32)
    for h in range(_J1B):
        out_ref[h * _NI:(h + 1) * _NI, :] = res[:, h * _D:(h + 1) * _D]


def _make_pair_table(G1, G2):
    g1t = jnp.repeat(
        jnp.transpose(G1, (3, 1, 0, 2)).reshape(16, _NI * 64), 2, axis=1)
    g2m = jnp.transpose(G2[:, :, :, 0], (1, 0, 2)).reshape(_NI, 32)
    return pl.pallas_call(
        _h12_body,
        grid=(_NI // _J1B,),
        in_specs=[
            pl.BlockSpec((_NI, 32), lambda i: (0, 0)),
            pl.BlockSpec((16, 128 * _J1B), lambda i: (0, i)),
        ],
        out_specs=pl.BlockSpec((_J1B * _NI, _D), lambda i: (i, 0)),
        out_shape=jax.ShapeDtypeStruct((_PAIRS, _D), jnp.float32),
    )(g2m, g1t)


def _sc_gather(xflat, table, bpw):
    B = xflat.shape[0]
    nch = bpw // _CHUNK
    mesh = plsc.VectorSubcoreMesh(core_axis_name="c", subcore_axis_name="s")

    @functools.partial(
        pl.kernel,
        mesh=mesh,
        out_type=jax.ShapeDtypeStruct((B, _D), jnp.float32),
        scratch_types=[
            pltpu.VMEM((bpw,), jnp.int32),
            pltpu.VMEM((bpw,), jnp.int32),
            pltpu.VMEM((_CHUNK, _D), jnp.float32),
            pltpu.VMEM((_CHUNK, _D), jnp.float32),
            pltpu.SemaphoreType.DMA,
            pltpu.SemaphoreType.DMA,
        ],
    )
    def gather(x_hbm, tab_hbm, out_hbm, xv, idxv, rows0, rows1, sem0, sem1):
        wid = lax.axis_index("s") * _NC + lax.axis_index("c")
        base = wid * bpw
        pltpu.sync_copy(x_hbm.at[pl.ds(base, bpw)], xv)

        def idx_body(i, carry):
            v = xv[pl.ds(i * 16, 16)]
            idxv[pl.ds(i * 16, 16)] = lax.rem(v, _PAIRS)
            return carry

        lax.fori_loop(0, bpw // 16, idx_body, 0)

        def pair_body(p, carry):
            c = p * 2
            d0 = pltpu.async_copy(
                tab_hbm.at[idxv.at[pl.ds(c * _CHUNK, _CHUNK)]], rows0, sem0)
            d1 = pltpu.async_copy(
                tab_hbm.at[idxv.at[pl.ds((c + 1) * _CHUNK, _CHUNK)]], rows1, sem1)
            d0.wait()
            pltpu.sync_copy(rows0, out_hbm.at[pl.ds(base + c * _CHUNK, _CHUNK)])
            d1.wait()
            pltpu.sync_copy(rows1, out_hbm.at[pl.ds(base + (c + 1) * _CHUNK, _CHUNK)])
            return carry

        lax.fori_loop(0, nch // 2, pair_body, 0)

    return gather(xflat, table)


def _contract_body(x_ref, hg_ref, g0_ref, out_ref):
    xr = x_ref[0]
    i0 = xr // _PAIRS
    iot = lax.broadcasted_iota(jnp.int32, (128, _TB), 0)
    oh = (iot == i0).astype(jnp.float32)
    a_t = jnp.dot(g0_ref[...], oh,
                  preferred_element_type=jnp.float32)
    hg_t = hg_ref[...].T
    q = _TB // 4
    rows = []
    for j in range(4):
        aj = a_t[:, j * q:(j + 1) * q]
        hj = hg_t[:, j * q:(j + 1) * q]
        for m0 in range(4):
            acc = aj[m0 * 16:m0 * 16 + 1, :] * hj[0:8, :]
            for r1 in range(1, 16):
                acc = acc + (aj[m0 * 16 + r1:m0 * 16 + r1 + 1, :]
                             * hj[r1 * 8:(r1 + 1) * 8, :])
            rows.append(acc)
    out_t = jnp.concatenate(rows, axis=0)
    out_ref[...] = out_t.T


def _contract(xflat, hg, g0m):
    B = xflat.shape[0]
    nb = B // _TB
    x3 = xflat.reshape(nb, 1, _TB)
    return pl.pallas_call(
        _contract_body,
        grid=(nb,),
        in_specs=[
            pl.BlockSpec((1, 1, _TB), lambda i: (i, 0, 0)),
            pl.BlockSpec((_TB, _D), lambda i: (i, 0)),
            pl.BlockSpec((64, 128), lambda i: (0, 0)),
        ],
        out_specs=pl.BlockSpec((_TB // 4, _D), lambda i: (i, 0)),
        out_shape=jax.ShapeDtypeStruct((B * 32 // _D, _D), jnp.float32),
    )(x3, hg, g0m)


def _g0_mat(G0):
    g = jnp.transpose(G0[0], (1, 2, 0)).reshape(64, _NI)
    return jnp.pad(g, ((0, 0), (0, 128 - _NI)))


def kernel(x, G0, G1, G2):
    xshape = x.shape
    xflat = x.reshape(-1)
    B = xflat.shape[0]
    bpw = B // _NW
    nb = B // _TB

    xperm = xflat.reshape(nb, _TB // 4, 4).transpose(0, 2, 1).reshape(-1)

    table = _make_pair_table(G1, G2)
    hg = _sc_gather(xperm, table, bpw)
    out = _contract(xperm, hg, _g0_mat(G0))
    return out.reshape(xshape + (32,))


# speedup vs baseline: 1.0007x; 1.0007x over previous
import functools

import jax
import jax.numpy as jnp
from jax import lax
from jax.experimental import pallas as pl
from jax.experimental.pallas import tpu as pltpu
from jax.experimental.pallas import tpu_sc as plsc

_NI = 100
_PAIRS = _NI * _NI
_D = 128

_NC = 2
_NS = 16
_NW = _NC * _NS

_CHUNK = 128
_BR = 64
_TB = _BR * 50


_J1B = 10


def _h12_body(g2m_ref, g1t_ref, out_ref):
    lhs = g2m_ref[...]
    w = _D * _J1B
    gc = jnp.repeat(g1t_ref[...], 2, axis=0)
    row_m2 = lax.broadcasted_iota(jnp.int32, (32, 1), 0) % 2
    col_m2 = lax.broadcasted_iota(jnp.int32, (1, w), 1) % 2
    rhs = jnp.where(row_m2 == col_m2, gc, 0.0)
    res = jnp.dot(lhs, rhs, preferred_element_type=jnp.float32)
    for h in range(_J1B):
        out_ref[h * _NI:(h + 1) * _NI, :] = res[:, h * _D:(h + 1) * _D]


def _make_pair_table(G1, G2):
    g1t = jnp.repeat(
        jnp.transpose(G1, (3, 1, 0, 2)).reshape(16, _NI * 64), 2, axis=1)
    g2m = jnp.transpose(G2[:, :, :, 0], (1, 0, 2)).reshape(_NI, 32)
    return pl.pallas_call(
        _h12_body,
        grid=(_NI // _J1B,),
        in_specs=[
            pl.BlockSpec((_NI, 32), lambda i: (0, 0)),
            pl.BlockSpec((16, 128 * _J1B), lambda i: (0, i)),
        ],
        out_specs=pl.BlockSpec((_J1B * _NI, _D), lambda i: (i, 0)),
        out_shape=jax.ShapeDtypeStruct((_PAIRS, _D), jnp.float32),
    )(g2m, g1t)


def _sc_gather(xflat, table, bpw):
    B = xflat.shape[0]
    nch = bpw // _CHUNK
    mesh = plsc.VectorSubcoreMesh(core_axis_name="c", subcore_axis_name="s")

    @functools.partial(
        pl.kernel,
        mesh=mesh,
        out_type=jax.ShapeDtypeStruct((B, _D), jnp.float32),
        scratch_types=[
            pltpu.VMEM((bpw,), jnp.int32),
            pltpu.VMEM((bpw,), jnp.int32),
            pltpu.VMEM((_CHUNK, _D), jnp.float32),
            pltpu.VMEM((_CHUNK, _D), jnp.float32),
            pltpu.SemaphoreType.DMA,
            pltpu.SemaphoreType.DMA,
        ],
    )
    def gather(x_hbm, tab_hbm, out_hbm, xv, idxv, rows0, rows1, sem0, sem1):
        wid = lax.axis_index("s") * _NC + lax.axis_index("c")
        base = wid * bpw
        pltpu.sync_copy(x_hbm.at[pl.ds(base, bpw)], xv)

        def idx_body(i, carry):
            v = xv[pl.ds(i * 16, 16)]
            idxv[pl.ds(i * 16, 16)] = lax.rem(v, _PAIRS)
            return carry

        lax.fori_loop(0, bpw // 16, idx_body, 0)

        def pair_body(p, carry):
            c = p * 2
            d0 = pltpu.async_copy(
                tab_hbm.at[idxv.at[pl.ds(c * _CHUNK, _CHUNK)]], rows0, sem0)
            d1 = pltpu.async_copy(
                tab_hbm.at[idxv.at[pl.ds((c + 1) * _CHUNK, _CHUNK)]], rows1, sem1)
            d0.wait()
            pltpu.sync_copy(rows0, out_hbm.at[pl.ds(base + c * _CHUNK, _CHUNK)])
            d1.wait()
            pltpu.sync_copy(rows1, out_hbm.at[pl.ds(base + (c + 1) * _CHUNK, _CHUNK)])
            return carry

        lax.fori_loop(0, nch // 2, pair_body, 0)
        if nch % 2:
            c = nch - 1
            d0 = pltpu.async_copy(
                tab_hbm.at[idxv.at[pl.ds(c * _CHUNK, _CHUNK)]], rows0, sem0)
            d0.wait()
            pltpu.sync_copy(rows0, out_hbm.at[pl.ds(base + c * _CHUNK, _CHUNK)])

    return gather(xflat, table)


def _contract_body(x_ref, hg_ref, g0_ref, out_ref):
    xr = x_ref[0]
    i0 = xr // _PAIRS
    iot = lax.broadcasted_iota(jnp.int32, (128, _TB), 0)
    oh = (iot == i0).astype(jnp.float32)
    a_t = jnp.dot(g0_ref[...], oh,
                  preferred_element_type=jnp.float32)
    hg_t = hg_ref[...].T
    q = _TB // 4
    rows = []
    for j in range(4):
        aj = a_t[:, j * q:(j + 1) * q]
        hj = hg_t[:, j * q:(j + 1) * q]
        for m0 in range(4):
            acc = aj[m0 * 16:m0 * 16 + 1, :] * hj[0:8, :]
            for r1 in range(1, 16):
                acc = acc + (aj[m0 * 16 + r1:m0 * 16 + r1 + 1, :]
                             * hj[r1 * 8:(r1 + 1) * 8, :])
            rows.append(acc)
    out_t = jnp.concatenate(rows, axis=0)
    out_ref[...] = out_t.T


def _contract(xflat, hg, g0m):
    B = xflat.shape[0]
    nb = B // _TB
    x3 = xflat.reshape(nb, 1, _TB)
    return pl.pallas_call(
        _contract_body,
        grid=(nb,),
        in_specs=[
            pl.BlockSpec((1, 1, _TB), lambda i: (i, 0, 0)),
            pl.BlockSpec((_TB, _D), lambda i: (i, 0)),
            pl.BlockSpec((64, 128), lambda i: (0, 0)),
        ],
        out_specs=pl.BlockSpec((_TB // 4, _D), lambda i: (i, 0)),
        out_shape=jax.ShapeDtypeStruct((B * 32 // _D, _D), jnp.float32),
    )(x3, hg, g0m)


def _g0_mat(G0):
    g = jnp.transpose(G0[0], (1, 2, 0)).reshape(64, _NI)
    return jnp.pad(g, ((0, 0), (0, 128 - _NI)))


def kernel(x, G0, G1, G2):
    xshape = x.shape
    xflat = x.reshape(-1)
    B = xflat.shape[0]
    bpw = B // _NW
    nb = B // _TB

    xperm = xflat.reshape(nb, _TB // 4, 4).transpose(0, 2, 1).reshape(-1)

    bh = B // 2
    xa, xb = xperm[:bh], xperm[bh:]
    g0m = _g0_mat(G0)
    table = _make_pair_table(G1, G2)
    hg_a = _sc_gather(xa, table, bh // _NW)
    hg_b = _sc_gather(xb, table, bh // _NW)
    out_a = _contract(xa, hg_a, g0m)
    out_b = _contract(xb, hg_b, g0m)
    out = jnp.concatenate([out_a, out_b], axis=0)
    return out.reshape(xshape + (32,))
